# baseline (speedup 1.0000x reference)
import functools

import jax
import jax.numpy as jnp
from jax import lax
from jax.experimental import pallas as pl
from jax.experimental.pallas import tpu as pltpu
from jax.experimental.pallas import tpu_sc as plsc

N_ROWS = 100000
N_FEAT = 128
N_SEG = 512

R_SC = 58880
R_TC = N_ROWS - R_SC

NC = 2
NS = 16
COLS = N_FEAT // NC
SEG_PER_TILE = N_SEG // NS
CHUNK = 128
NBUF = 4
PER_TILE = R_SC // NS
NFULL = PER_TILE // CHUNK
REM = PER_TILE - NFULL * CHUNK

BLK = 2048
NBLK = -(-R_TC // BLK)
R_TC_PAD = NBLK * BLK

_mesh = plsc.VectorSubcoreMesh(core_axis_name="c", subcore_axis_name="s")


@functools.partial(
    pl.kernel,
    out_type=jax.ShapeDtypeStruct((N_SEG, N_FEAT), jnp.float32),
    mesh=_mesh,
    scratch_types=[
        pltpu.VMEM_SHARED((N_SEG, COLS), jnp.float32),
        pltpu.VMEM((NBUF, CHUNK, COLS), jnp.float32),
        pltpu.VMEM((NBUF, CHUNK), jnp.int32),
    ] + [pltpu.SemaphoreType.DMA] * (2 * NBUF),
    compiler_params=pltpu.CompilerParams(use_tc_tiling_on_sc=False),
)
def _sc_segment_sum(x_hbm, b_hbm, out_hbm, acc, xbuf, idxbuf, *sems):
    semx = sems[0:NBUF]
    semi = sems[NBUF:2 * NBUF]
    c = lax.axis_index("c")
    s = lax.axis_index("s")
    col0 = c * COLS
    base = R_TC + s * PER_TILE

    zvec = jnp.zeros((16,), jnp.float32)
    def _zrow(i, carry):
        for q in range(COLS // 16):
            xbuf[0, i, 16 * q:16 * q + 16] = zvec
        return carry
    lax.fori_loop(0, CHUNK, _zrow, 0)
    zidx = jnp.zeros((16,), jnp.int32)
    for q in range(CHUNK // 16):
        idxbuf[0, 16 * q:16 * q + 16] = zidx

    pltpu.sync_copy(xbuf.at[0, pl.ds(0, SEG_PER_TILE)],
                    acc.at[pl.ds(s * SEG_PER_TILE, SEG_PER_TILE)])
    plsc.subcore_barrier()

    def load_descs(jj, b):
        start = base + jj * CHUNK
        return (
            pltpu.make_async_copy(b_hbm.at[pl.ds(start, CHUNK)],
                                  idxbuf.at[b], semi[b]),
            pltpu.make_async_copy(
                x_hbm.at[pl.ds(start, CHUNK), pl.ds(col0, COLS)],
                xbuf.at[b], semx[b]),
        )

    def scatter(b):
        pltpu.sync_copy(xbuf.at[b], acc.at[idxbuf.at[b]], add=True)

    rem_start = base + NFULL * CHUNK
    pltpu.sync_copy(b_hbm.at[pl.ds(rem_start, REM)],
                    idxbuf.at[0, pl.ds(0, REM)])
    pltpu.sync_copy(x_hbm.at[pl.ds(rem_start, REM), pl.ds(col0, COLS)],
                    xbuf.at[0, pl.ds(0, REM)])
    pltpu.sync_copy(xbuf.at[0], acc.at[idxbuf.at[0]], add=True)

    for b in range(NBUF - 1):
        for d in load_descs(b, b):
            d.start()

    def step(jj, b):
        for d in load_descs(jj, b):
            d.wait()
        scatter(b)

        @pl.when(jj + NBUF - 1 < NFULL)
        def _():
            for d in load_descs(jj + NBUF - 1, (b + NBUF - 1) % NBUF):
                d.start()

    def pipe(j, carry):
        for b in range(NBUF):
            step(NBUF * j + b, b)
        return carry
    lax.fori_loop(0, NFULL // NBUF, pipe, 0)
    for jj in range(NFULL - NFULL % NBUF, NFULL):
        step(jj, jj % NBUF)

    plsc.subcore_barrier()
    pltpu.sync_copy(acc.at[pl.ds(s * SEG_PER_TILE, SEG_PER_TILE)],
                    out_hbm.at[pl.ds(s * SEG_PER_TILE, SEG_PER_TILE),
                               pl.ds(col0, COLS)])


def _tc_body(bid_ref, x_ref, out_ref):
    pid = pl.program_id(0)
    ids = bid_ref[0, 0, :]
    one_hot = (ids[:, None]
               == lax.broadcasted_iota(jnp.int32, (BLK, N_SEG), 1)
               ).astype(jnp.float32)
    part = lax.dot_general(one_hot, x_ref[...], (((0,), (0,)), ((), ())),
                           preferred_element_type=jnp.float32)

    @pl.when(pid == 0)
    def _():
        out_ref[...] = part

    @pl.when(pid != 0)
    def _():
        out_ref[...] = out_ref[...] + part


_tc_call = pl.pallas_call(
    _tc_body,
    grid=(NBLK,),
    in_specs=[pl.BlockSpec((1, 1, BLK), lambda j: (j, 0, 0)),
              pl.BlockSpec((BLK, N_FEAT), lambda j: (j, 0))],
    out_specs=pl.BlockSpec((N_SEG, N_FEAT), lambda j: (0, 0)),
    out_shape=jax.ShapeDtypeStruct((N_SEG, N_FEAT), jnp.float32),
)


def _add_body(a_ref, b_ref, o_ref):
    o_ref[...] = a_ref[...] + b_ref[...]


_add_call = pl.pallas_call(
    _add_body,
    out_shape=jax.ShapeDtypeStruct((N_SEG, N_FEAT), jnp.float32),
)


def kernel(x, batch):
    batch32 = batch.astype(jnp.int32)
    out_sc = _sc_segment_sum(x, batch32)
    ids_tc = jnp.concatenate(
        [batch32[:R_TC],
         jnp.full((R_TC_PAD - R_TC,), -1, jnp.int32)]).reshape(NBLK, 1, BLK)
    out_tc = _tc_call(ids_tc, x)
    return _add_call(out_sc, out_tc)


# speedup vs baseline: 1.0795x; 1.0795x over previous
import functools

import jax
import jax.numpy as jnp
from jax import lax
from jax.experimental import pallas as pl
from jax.experimental.pallas import tpu as pltpu
from jax.experimental.pallas import tpu_sc as plsc

N_ROWS = 100000
N_FEAT = 128
N_SEG = 512

R_SC = 64640
R_TC = N_ROWS - R_SC

NC = 2
NS = 16
COLS = N_FEAT // NC
SEG_PER_TILE = N_SEG // NS
CHUNK = 128
NBUF = 4
PER_TILE = R_SC // NS
NFULL = PER_TILE // CHUNK
REM = PER_TILE - NFULL * CHUNK

BLK = 2048
NBLK = -(-R_TC // BLK)
R_TC_PAD = NBLK * BLK

_mesh = plsc.VectorSubcoreMesh(core_axis_name="c", subcore_axis_name="s")


@functools.partial(
    pl.kernel,
    out_type=jax.ShapeDtypeStruct((N_SEG, N_FEAT), jnp.float32),
    mesh=_mesh,
    scratch_types=[
        pltpu.VMEM_SHARED((N_SEG, COLS), jnp.float32),
        pltpu.VMEM((NBUF, CHUNK, COLS), jnp.float32),
        pltpu.VMEM((NBUF, CHUNK), jnp.int32),
    ] + [pltpu.SemaphoreType.DMA] * (2 * NBUF),
    compiler_params=pltpu.CompilerParams(use_tc_tiling_on_sc=False),
)
def _sc_segment_sum(x_hbm, b_hbm, out_hbm, acc, xbuf, idxbuf, *sems):
    semx = sems[0:NBUF]
    semi = sems[NBUF:2 * NBUF]
    c = lax.axis_index("c")
    s = lax.axis_index("s")
    col0 = c * COLS
    base = R_TC + s * PER_TILE

    zvec = jnp.zeros((16,), jnp.float32)
    def _zrow(i, carry):
        for q in range(COLS // 16):
            xbuf[0, i, 16 * q:16 * q + 16] = zvec
        return carry
    lax.fori_loop(0, CHUNK, _zrow, 0)
    zidx = jnp.zeros((16,), jnp.int32)
    for q in range(CHUNK // 16):
        idxbuf[0, 16 * q:16 * q + 16] = zidx

    pltpu.sync_copy(xbuf.at[0, pl.ds(0, SEG_PER_TILE)],
                    acc.at[pl.ds(s * SEG_PER_TILE, SEG_PER_TILE)])
    plsc.subcore_barrier()

    def load_descs(jj, b):
        start = base + jj * CHUNK
        return (
            pltpu.make_async_copy(b_hbm.at[pl.ds(start, CHUNK)],
                                  idxbuf.at[b], semi[b]),
            pltpu.make_async_copy(
                x_hbm.at[pl.ds(start, CHUNK), pl.ds(col0, COLS)],
                xbuf.at[b], semx[b]),
        )

    def scatter(b):
        pltpu.sync_copy(xbuf.at[b], acc.at[idxbuf.at[b]], add=True)

    rem_start = base + NFULL * CHUNK
    pltpu.sync_copy(b_hbm.at[pl.ds(rem_start, REM)],
                    idxbuf.at[0, pl.ds(0, REM)])
    pltpu.sync_copy(x_hbm.at[pl.ds(rem_start, REM), pl.ds(col0, COLS)],
                    xbuf.at[0, pl.ds(0, REM)])
    pltpu.sync_copy(xbuf.at[0], acc.at[idxbuf.at[0]], add=True)

    for b in range(NBUF - 1):
        for d in load_descs(b, b):
            d.start()

    def step(jj, b):
        for d in load_descs(jj, b):
            d.wait()
        scatter(b)

        @pl.when(jj + NBUF - 1 < NFULL)
        def _():
            for d in load_descs(jj + NBUF - 1, (b + NBUF - 1) % NBUF):
                d.start()

    def pipe(j, carry):
        for b in range(NBUF):
            step(NBUF * j + b, b)
        return carry
    lax.fori_loop(0, NFULL // NBUF, pipe, 0)
    for jj in range(NFULL - NFULL % NBUF, NFULL):
        step(jj, jj % NBUF)

    plsc.subcore_barrier()
    pltpu.sync_copy(acc.at[pl.ds(s * SEG_PER_TILE, SEG_PER_TILE)],
                    out_hbm.at[pl.ds(s * SEG_PER_TILE, SEG_PER_TILE),
                               pl.ds(col0, COLS)])


def _tc_body(bid_ref, x_ref, out_ref):
    pid = pl.program_id(0)
    ids = bid_ref[0, 0, :]
    one_hot = (ids[:, None]
               == lax.broadcasted_iota(jnp.int32, (BLK, N_SEG), 1)
               ).astype(jnp.float32)
    part = lax.dot_general(one_hot, x_ref[...], (((0,), (0,)), ((), ())),
                           preferred_element_type=jnp.float32)

    @pl.when(pid == 0)
    def _():
        out_ref[...] = part

    @pl.when(pid != 0)
    def _():
        out_ref[...] = out_ref[...] + part


_tc_call = pl.pallas_call(
    _tc_body,
    grid=(NBLK,),
    in_specs=[pl.BlockSpec((1, 1, BLK), lambda j: (j, 0, 0)),
              pl.BlockSpec((BLK, N_FEAT), lambda j: (j, 0))],
    out_specs=pl.BlockSpec((N_SEG, N_FEAT), lambda j: (0, 0)),
    out_shape=jax.ShapeDtypeStruct((N_SEG, N_FEAT), jnp.float32),
)


def _add_body(a_ref, b_ref, o_ref):
    o_ref[...] = a_ref[...] + b_ref[...]


_add_call = pl.pallas_call(
    _add_body,
    out_shape=jax.ShapeDtypeStruct((N_SEG, N_FEAT), jnp.float32),
)


def kernel(x, batch):
    batch32 = batch.astype(jnp.int32)
    out_sc = _sc_segment_sum(x, batch32)
    ids_tc = jnp.concatenate(
        [batch32[:R_TC],
         jnp.full((R_TC_PAD - R_TC,), -1, jnp.int32)]).reshape(NBLK, 1, BLK)
    out_tc = _tc_call(ids_tc, x)
    return _add_call(out_sc, out_tc)


# speedup vs baseline: 1.1091x; 1.0274x over previous
import functools

import jax
import jax.numpy as jnp
from jax import lax
from jax.experimental import pallas as pl
from jax.experimental.pallas import tpu as pltpu
from jax.experimental.pallas import tpu_sc as plsc

N_ROWS = 100000
N_FEAT = 128
N_SEG = 512

R_SC = 65280
R_TC = N_ROWS - R_SC

NC = 2
NS = 16
COLS = N_FEAT // NC
SEG_PER_TILE = N_SEG // NS
CHUNK = 128
NBUF = 4
PER_TILE = R_SC // NS
NFULL = PER_TILE // CHUNK
REM = PER_TILE - NFULL * CHUNK

BLK = 2048
NBLK = -(-R_TC // BLK)
R_TC_PAD = NBLK * BLK

_mesh = plsc.VectorSubcoreMesh(core_axis_name="c", subcore_axis_name="s")


@functools.partial(
    pl.kernel,
    out_type=jax.ShapeDtypeStruct((N_SEG, N_FEAT), jnp.float32),
    mesh=_mesh,
    scratch_types=[
        pltpu.VMEM_SHARED((N_SEG, COLS), jnp.float32),
        pltpu.VMEM((NBUF, CHUNK, COLS), jnp.float32),
        pltpu.VMEM((NBUF, CHUNK), jnp.int32),
    ] + [pltpu.SemaphoreType.DMA] * (2 * NBUF),
    compiler_params=pltpu.CompilerParams(use_tc_tiling_on_sc=False),
)
def _sc_segment_sum(x_hbm, b_hbm, out_hbm, acc, xbuf, idxbuf, *sems):
    semx = sems[0:NBUF]
    semi = sems[NBUF:2 * NBUF]
    c = lax.axis_index("c")
    s = lax.axis_index("s")
    col0 = c * COLS
    base = R_TC + s * PER_TILE

    zvec = jnp.zeros((16,), jnp.float32)
    def _zrow(i, carry):
        for q in range(COLS // 16):
            xbuf[0, i, 16 * q:16 * q + 16] = zvec
        return carry
    lax.fori_loop(0, CHUNK, _zrow, 0)
    zidx = jnp.zeros((16,), jnp.int32)
    for q in range(CHUNK // 16):
        idxbuf[0, 16 * q:16 * q + 16] = zidx

    pltpu.sync_copy(xbuf.at[0, pl.ds(0, SEG_PER_TILE)],
                    acc.at[pl.ds(s * SEG_PER_TILE, SEG_PER_TILE)])
    plsc.subcore_barrier()

    def load_descs(jj, b):
        start = base + jj * CHUNK
        return (
            pltpu.make_async_copy(b_hbm.at[pl.ds(start, CHUNK)],
                                  idxbuf.at[b], semi[b]),
            pltpu.make_async_copy(
                x_hbm.at[pl.ds(start, CHUNK), pl.ds(col0, COLS)],
                xbuf.at[b], semx[b]),
        )

    def scatter(b):
        pltpu.sync_copy(xbuf.at[b], acc.at[idxbuf.at[b]], add=True)

    rem_start = base + NFULL * CHUNK
    pltpu.sync_copy(b_hbm.at[pl.ds(rem_start, REM)],
                    idxbuf.at[0, pl.ds(0, REM)])
    pltpu.sync_copy(x_hbm.at[pl.ds(rem_start, REM), pl.ds(col0, COLS)],
                    xbuf.at[0, pl.ds(0, REM)])
    pltpu.sync_copy(xbuf.at[0], acc.at[idxbuf.at[0]], add=True)

    for b in range(NBUF - 1):
        for d in load_descs(b, b):
            d.start()

    def step(jj, b):
        for d in load_descs(jj, b):
            d.wait()
        scatter(b)

        @pl.when(jj + NBUF - 1 < NFULL)
        def _():
            for d in load_descs(jj + NBUF - 1, (b + NBUF - 1) % NBUF):
                d.start()

    def pipe(j, carry):
        for b in range(NBUF):
            step(NBUF * j + b, b)
        return carry
    lax.fori_loop(0, NFULL // NBUF, pipe, 0)
    for jj in range(NFULL - NFULL % NBUF, NFULL):
        step(jj, jj % NBUF)

    plsc.subcore_barrier()
    pltpu.sync_copy(acc.at[pl.ds(s * SEG_PER_TILE, SEG_PER_TILE)],
                    out_hbm.at[pl.ds(s * SEG_PER_TILE, SEG_PER_TILE),
                               pl.ds(col0, COLS)])


def _tc_body(bid_ref, x_ref, out_ref):
    pid = pl.program_id(0)
    ids = bid_ref[0, 0, :]
    one_hot = (ids[:, None]
               == lax.broadcasted_iota(jnp.int32, (BLK, N_SEG), 1)
               ).astype(jnp.float32)
    part = lax.dot_general(one_hot, x_ref[...], (((0,), (0,)), ((), ())),
                           preferred_element_type=jnp.float32)

    @pl.when(pid == 0)
    def _():
        out_ref[...] = part

    @pl.when(pid != 0)
    def _():
        out_ref[...] = out_ref[...] + part


_tc_call = pl.pallas_call(
    _tc_body,
    grid=(NBLK,),
    in_specs=[pl.BlockSpec((1, 1, BLK), lambda j: (j, 0, 0)),
              pl.BlockSpec((BLK, N_FEAT), lambda j: (j, 0))],
    out_specs=pl.BlockSpec((N_SEG, N_FEAT), lambda j: (0, 0)),
    out_shape=jax.ShapeDtypeStruct((N_SEG, N_FEAT), jnp.float32),
)


def _add_body(a_ref, b_ref, o_ref):
    o_ref[...] = a_ref[...] + b_ref[...]


_add_call = pl.pallas_call(
    _add_body,
    out_shape=jax.ShapeDtypeStruct((N_SEG, N_FEAT), jnp.float32),
)


def kernel(x, batch):
    batch32 = batch.astype(jnp.int32)
    out_sc = _sc_segment_sum(x, batch32)
    ids_tc = jnp.concatenate(
        [batch32[:R_TC],
         jnp.full((R_TC_PAD - R_TC,), -1, jnp.int32)]).reshape(NBLK, 1, BLK)
    out_tc = _tc_call(ids_tc, x)
    return _add_call(out_sc, out_tc)
